# Initial kernel scaffold; baseline (speedup 1.0000x reference)
import functools

import jax
import jax.numpy as jnp
from jax import lax
from jax.experimental import pallas as pl
from jax.experimental.pallas import tpu as pltpu
from jax.experimental.pallas import tpu_sc as plsc

N_NODES = 100000
DIM = 128
NUM_GRAPHS = 512

NC = 2
NS = 16
NW = NC * NS

BLK = 128
NFULL = N_NODES // BLK
TAIL = N_NODES - NFULL * BLK
ITERS = (NFULL + NW - 1) // NW
CW = 16
GROWS = NUM_GRAPHS // NS

_mesh = plsc.VectorSubcoreMesh(core_axis_name="c", subcore_axis_name="s")


@functools.partial(
    pl.kernel,
    out_type=(
        jax.ShapeDtypeStruct((NC, NUM_GRAPHS, DIM), jnp.float32),
        jax.ShapeDtypeStruct((NC, NUM_GRAPHS, CW), jnp.float32),
    ),
    mesh=_mesh,
    scratch_types=[
        pltpu.VMEM((BLK, DIM), jnp.float32),
        pltpu.VMEM((BLK,), jnp.int32),
        pltpu.VMEM((BLK, CW), jnp.float32),
        pltpu.VMEM((TAIL, DIM), jnp.float32),
        pltpu.VMEM((TAIL,), jnp.int32),
        pltpu.VMEM((TAIL, CW), jnp.float32),
        pltpu.VMEM((GROWS, DIM), jnp.float32),
        pltpu.VMEM((GROWS, CW), jnp.float32),
        pltpu.VMEM_SHARED((NUM_GRAPHS, DIM), jnp.float32),
        pltpu.VMEM_SHARED((NUM_GRAPHS, CW), jnp.float32),
    ],
)
def _segment_pool(feat_hbm, ids_hbm, psums_hbm, pcnts_hbm,
                  feat_v, ids_v, ones_v, feat_t, ids_t, ones_t,
                  zrow_v, zcnt_v, sums_sp, cnts_sp):
    cid = lax.axis_index("c")
    sid = lax.axis_index("s")
    wid = cid * NS + sid

    zeros16 = jnp.zeros((16,), jnp.float32)
    ones16 = jnp.ones((16,), jnp.float32)

    def _fill_zrow(i, _):
        zrow_v[i // (DIM // 16), pl.ds((i % (DIM // 16)) * 16, 16)] = zeros16
        return 0
    lax.fori_loop(0, GROWS * (DIM // 16), _fill_zrow, 0)

    def _fill_zcnt(i, _):
        zcnt_v[i, pl.ds(0, 16)] = zeros16
        return 0
    lax.fori_loop(0, GROWS, _fill_zcnt, 0)

    def _fill_ones(i, _):
        ones_v[i, pl.ds(0, 16)] = ones16
        return 0
    lax.fori_loop(0, BLK, _fill_ones, 0)

    def _fill_ones_t(i, _):
        ones_t[i, pl.ds(0, 16)] = ones16
        return 0
    lax.fori_loop(0, TAIL, _fill_ones_t, 0)

    pltpu.sync_copy(zrow_v, sums_sp.at[pl.ds(sid * GROWS, GROWS)])
    pltpu.sync_copy(zcnt_v, cnts_sp.at[pl.ds(sid * GROWS, GROWS)])
    plsc.subcore_barrier()

    def _body(i, _):
        blk = i * NW + wid

        @pl.when(blk < NFULL)
        def _():
            base = blk * BLK
            pltpu.sync_copy(feat_hbm.at[pl.ds(base, BLK)], feat_v)
            pltpu.sync_copy(ids_hbm.at[pl.ds(base, BLK)], ids_v)
            pltpu.sync_copy(feat_v, sums_sp.at[ids_v], add=True)
            pltpu.sync_copy(ones_v, cnts_sp.at[ids_v], add=True)
        return 0

    lax.fori_loop(0, ITERS, _body, 0)

    @pl.when(wid == NW - 1)
    def _():
        base = NFULL * BLK
        pltpu.sync_copy(feat_hbm.at[pl.ds(base, TAIL)], feat_t)
        pltpu.sync_copy(ids_hbm.at[pl.ds(base, TAIL)], ids_t)
        pltpu.sync_copy(feat_t, sums_sp.at[ids_t], add=True)
        pltpu.sync_copy(ones_t, cnts_sp.at[ids_t], add=True)

    plsc.subcore_barrier()

    pltpu.sync_copy(sums_sp.at[pl.ds(sid * GROWS, GROWS)],
                    psums_hbm.at[cid, pl.ds(sid * GROWS, GROWS)])
    pltpu.sync_copy(cnts_sp.at[pl.ds(sid * GROWS, GROWS)],
                    pcnts_hbm.at[cid, pl.ds(sid * GROWS, GROWS)])


def _head_body(ps_ref, pc_ref, w_ref, b_ref, out_ref):
    sums = ps_ref[0] + ps_ref[1]
    cnt = (pc_ref[0] + pc_ref[1])[:, 0:1]
    emb = sums / jnp.maximum(cnt, 1.0)
    out_ref[...] = (
        jnp.dot(emb, w_ref[...], preferred_element_type=jnp.float32)
        + b_ref[...]
    )


def _head(psums, pcnts, W, b2d):
    return pl.pallas_call(
        _head_body,
        out_shape=jax.ShapeDtypeStruct((NUM_GRAPHS, DIM), jnp.float32),
    )(psums, pcnts, W, b2d)


@jax.jit
def kernel(node_feature, batch_ids, graph_label, W, b):
    ids = batch_ids.astype(jnp.int32)
    psums, pcnts = _segment_pool(node_feature, ids)
    pred = _head(psums, pcnts, W, b.reshape(1, DIM))
    return (pred, graph_label)


# speedup vs baseline: 4.5560x; 4.5560x over previous
import functools

import jax
import jax.numpy as jnp
from jax import lax
from jax.experimental import pallas as pl
from jax.experimental.pallas import tpu as pltpu
from jax.experimental.pallas import tpu_sc as plsc

N_NODES = 100000
DIM = 128
NUM_GRAPHS = 512

NC = 2
NS = 16
NW = NC * NS

BLK = 128
NFULL = N_NODES // BLK
TAIL = N_NODES - NFULL * BLK
ITERS = (NFULL + NW - 1) // NW
CW = 128
GROWS = NUM_GRAPHS // NS

_mesh = plsc.VectorSubcoreMesh(core_axis_name="c", subcore_axis_name="s")


@functools.partial(
    pl.kernel,
    out_type=(
        jax.ShapeDtypeStruct((NC, NUM_GRAPHS, DIM), jnp.float32),
        jax.ShapeDtypeStruct((NC, NUM_GRAPHS, CW), jnp.float32),
    ),
    mesh=_mesh,
    scratch_types=[
        pltpu.VMEM((BLK, DIM), jnp.float32),
        pltpu.VMEM((BLK,), jnp.int32),
        pltpu.VMEM((BLK, CW), jnp.float32),
        pltpu.VMEM((TAIL, DIM), jnp.float32),
        pltpu.VMEM((TAIL,), jnp.int32),
        pltpu.VMEM((TAIL, CW), jnp.float32),
        pltpu.VMEM_SHARED((NUM_GRAPHS, DIM), jnp.float32),
        pltpu.VMEM_SHARED((NUM_GRAPHS, CW), jnp.float32),
    ],
)
def _segment_pool(feat_hbm, ids_hbm, zsum_hbm, ones_hbm,
                  psums_hbm, pcnts_hbm,
                  feat_v, ids_v, ones_v, feat_t, ids_t, ones_t,
                  sums_sp, cnts_sp):
    cid = lax.axis_index("c")
    sid = lax.axis_index("s")
    wid = cid * NS + sid

    pltpu.sync_copy(ones_hbm, ones_v)
    pltpu.sync_copy(ones_hbm.at[pl.ds(0, TAIL)], ones_t)
    pltpu.sync_copy(zsum_hbm.at[pl.ds(sid * GROWS, GROWS)],
                    sums_sp.at[pl.ds(sid * GROWS, GROWS)])
    pltpu.sync_copy(zsum_hbm.at[pl.ds(sid * GROWS, GROWS)],
                    cnts_sp.at[pl.ds(sid * GROWS, GROWS)])
    plsc.subcore_barrier()

    def _body(i, _):
        blk = i * NW + wid

        @pl.when(blk < NFULL)
        def _():
            base = blk * BLK
            pltpu.sync_copy(feat_hbm.at[pl.ds(base, BLK)], feat_v)
            pltpu.sync_copy(ids_hbm.at[pl.ds(base, BLK)], ids_v)
            pltpu.sync_copy(feat_v, sums_sp.at[ids_v], add=True)
            pltpu.sync_copy(ones_v, cnts_sp.at[ids_v], add=True)
        return 0

    lax.fori_loop(0, ITERS, _body, 0)

    @pl.when(wid == NW - 1)
    def _():
        base = NFULL * BLK
        pltpu.sync_copy(feat_hbm.at[pl.ds(base, TAIL)], feat_t)
        pltpu.sync_copy(ids_hbm.at[pl.ds(base, TAIL)], ids_t)
        pltpu.sync_copy(feat_t, sums_sp.at[ids_t], add=True)
        pltpu.sync_copy(ones_t, cnts_sp.at[ids_t], add=True)

    plsc.subcore_barrier()

    pltpu.sync_copy(sums_sp.at[pl.ds(sid * GROWS, GROWS)],
                    psums_hbm.at[cid, pl.ds(sid * GROWS, GROWS)])
    pltpu.sync_copy(cnts_sp.at[pl.ds(sid * GROWS, GROWS)],
                    pcnts_hbm.at[cid, pl.ds(sid * GROWS, GROWS)])


def _head_body(ps_ref, pc_ref, w_ref, b_ref, out_ref):
    sums = ps_ref[0] + ps_ref[1]
    cnt = (pc_ref[0] + pc_ref[1])[:, 0:1]
    emb = sums / jnp.maximum(cnt, 1.0)
    out_ref[...] = (
        jnp.dot(emb, w_ref[...], preferred_element_type=jnp.float32)
        + b_ref[...]
    )


def _head(psums, pcnts, W, b2d):
    return pl.pallas_call(
        _head_body,
        out_shape=jax.ShapeDtypeStruct((NUM_GRAPHS, DIM), jnp.float32),
    )(psums, pcnts, W, b2d)


@jax.jit
def kernel(node_feature, batch_ids, graph_label, W, b):
    ids = batch_ids.astype(jnp.int32)
    zsum = jnp.zeros((NUM_GRAPHS, DIM), jnp.float32)
    ones = jnp.ones((BLK, CW), jnp.float32)
    psums, pcnts = _segment_pool(node_feature, ids, zsum, ones)
    pred = _head(psums, pcnts, W, b.reshape(1, DIM))
    return (pred, graph_label)


# speedup vs baseline: 5.6190x; 1.2333x over previous
import functools

import jax
import jax.numpy as jnp
from jax import lax
from jax.experimental import pallas as pl
from jax.experimental.pallas import tpu as pltpu
from jax.experimental.pallas import tpu_sc as plsc

N_NODES = 100000
DIM = 128
NUM_GRAPHS = 512

NC = 2
NS = 16
NW = NC * NS

BLK = 128
NFULL = N_NODES // BLK
TAIL = N_NODES - NFULL * BLK
ITERS = (NFULL + NW - 1) // NW
GROWS = NUM_GRAPHS // NS

_mesh = plsc.VectorSubcoreMesh(core_axis_name="c", subcore_axis_name="s")


@functools.partial(
    pl.kernel,
    out_type=(
        jax.ShapeDtypeStruct((NC, NUM_GRAPHS, DIM), jnp.float32),
        jax.ShapeDtypeStruct((NW, NUM_GRAPHS), jnp.float32),
    ),
    mesh=_mesh,
    compiler_params=pltpu.CompilerParams(needs_layout_passes=False),
    scratch_types=[
        pltpu.VMEM((BLK, DIM), jnp.float32),
        pltpu.VMEM((BLK,), jnp.int32),
        pltpu.VMEM((TAIL, DIM), jnp.float32),
        pltpu.VMEM((TAIL,), jnp.int32),
        pltpu.VMEM((NUM_GRAPHS,), jnp.float32),
        pltpu.VMEM_SHARED((NUM_GRAPHS, DIM), jnp.float32),
    ],
)
def _segment_pool(feat_hbm, ids_hbm, zsum_hbm, zl_hbm,
                  psums_hbm, lout_hbm,
                  feat_v, ids_v, feat_t, ids_t, ltab_v, sums_sp):
    cid = lax.axis_index("c")
    sid = lax.axis_index("s")
    wid = cid * NS + sid

    iota16 = lax.iota(jnp.int32, 16)

    pltpu.sync_copy(zl_hbm, ltab_v)
    pltpu.sync_copy(zsum_hbm.at[pl.ds(sid * GROWS, GROWS)],
                    sums_sp.at[pl.ds(sid * GROWS, GROWS)])
    plsc.subcore_barrier()

    def _positions(base, j):
        return (base + (j * 16 + 1) + iota16).astype(jnp.float32)

    def _body(i, _):
        blk = i * NW + wid

        @pl.when(blk < NFULL)
        def _():
            base = blk * BLK
            pltpu.sync_copy(feat_hbm.at[pl.ds(base, BLK)], feat_v)
            pltpu.sync_copy(ids_hbm.at[pl.ds(base, BLK)], ids_v)
            pltpu.sync_copy(feat_v, sums_sp.at[ids_v], add=True)
            for j in range(BLK // 16):
                idx = ids_v[pl.ds(j * 16, 16)]
                plsc.store_scatter(ltab_v, [idx], _positions(base, j))
        return 0

    lax.fori_loop(0, ITERS, _body, 0)

    @pl.when(wid == NW - 1)
    def _():
        base = NFULL * BLK
        pltpu.sync_copy(feat_hbm.at[pl.ds(base, TAIL)], feat_t)
        pltpu.sync_copy(ids_hbm.at[pl.ds(base, TAIL)], ids_t)
        pltpu.sync_copy(feat_t, sums_sp.at[ids_t], add=True)
        for j in range(TAIL // 16):
            idx = ids_t[pl.ds(j * 16, 16)]
            plsc.store_scatter(ltab_v, [idx], _positions(base, j))

    plsc.subcore_barrier()

    pltpu.sync_copy(sums_sp.at[pl.ds(sid * GROWS, GROWS)],
                    psums_hbm.at[cid, pl.ds(sid * GROWS, GROWS)])
    pltpu.sync_copy(ltab_v, lout_hbm.at[wid])


def _head_body(ps_ref, lt_ref, w_ref, b_ref, out_ref):
    lpos = jnp.max(lt_ref[...], axis=0)
    gi = lax.broadcasted_iota(jnp.int32, (NUM_GRAPHS, NUM_GRAPHS), 0)
    gj = lax.broadcasted_iota(jnp.int32, (NUM_GRAPHS, NUM_GRAPHS), 1)
    prev = jnp.max(jnp.where(gi < gj, lpos[:, None], 0.0), axis=0)
    cnt = jnp.maximum(lpos - prev, 1.0)[:, None]
    emb = (ps_ref[0] + ps_ref[1]) / cnt
    out_ref[...] = (
        jnp.dot(emb, w_ref[...], preferred_element_type=jnp.float32)
        + b_ref[...]
    )


def _head(psums, lout, W, b2d):
    return pl.pallas_call(
        _head_body,
        out_shape=jax.ShapeDtypeStruct((NUM_GRAPHS, DIM), jnp.float32),
    )(psums, lout, W, b2d)


@jax.jit
def kernel(node_feature, batch_ids, graph_label, W, b):
    ids = batch_ids.astype(jnp.int32)
    zsum = jnp.zeros((NUM_GRAPHS, DIM), jnp.float32)
    zl = jnp.zeros((NUM_GRAPHS,), jnp.float32)
    psums, lout = _segment_pool(node_feature, ids, zsum, zl)
    pred = _head(psums, lout, W, b.reshape(1, DIM))
    return (pred, graph_label)


# speedup vs baseline: 8.2423x; 1.4668x over previous
import functools

import jax
import jax.numpy as jnp
from jax import lax
from jax.experimental import pallas as pl
from jax.experimental.pallas import tpu as pltpu
from jax.experimental.pallas import tpu_sc as plsc

N_NODES = 100000
DIM = 128
NUM_GRAPHS = 512

NC = 2
NS = 16
NW = NC * NS

BLK = 128
NFULL = N_NODES // BLK
TAIL = N_NODES - NFULL * BLK
ITERS = (NFULL + NW - 1) // NW
GROWS = NUM_GRAPHS // NS

_mesh = plsc.VectorSubcoreMesh(core_axis_name="c", subcore_axis_name="s")


@functools.partial(
    pl.kernel,
    out_type=(
        jax.ShapeDtypeStruct((NC, NUM_GRAPHS, DIM), jnp.float32),
        jax.ShapeDtypeStruct((NW, NUM_GRAPHS), jnp.float32),
    ),
    mesh=_mesh,
    compiler_params=pltpu.CompilerParams(needs_layout_passes=False),
    scratch_types=[
        pltpu.VMEM((BLK, DIM), jnp.float32),
        pltpu.VMEM((BLK, DIM), jnp.float32),
        pltpu.VMEM((BLK,), jnp.int32),
        pltpu.VMEM((BLK,), jnp.int32),
        pltpu.VMEM((TAIL, DIM), jnp.float32),
        pltpu.VMEM((TAIL,), jnp.int32),
        pltpu.VMEM((NUM_GRAPHS,), jnp.float32),
        pltpu.VMEM_SHARED((NUM_GRAPHS, DIM), jnp.float32),
        pltpu.SemaphoreType.DMA,
        pltpu.SemaphoreType.DMA,
        pltpu.SemaphoreType.DMA,
        pltpu.SemaphoreType.DMA,
    ],
)
def _segment_pool(feat_hbm, ids_hbm, zsum_hbm, zl_hbm,
                  psums_hbm, lout_hbm,
                  feat_a, feat_b, ids_a, ids_b, feat_t, ids_t, ltab_v,
                  sums_sp, fsem0, fsem1, isem0, isem1):
    cid = lax.axis_index("c")
    sid = lax.axis_index("s")
    wid = cid * NS + sid
    fsems = (fsem0, fsem1)
    isems = (isem0, isem1)
    feats = (feat_a, feat_b)
    idss = (ids_a, ids_b)

    iota16 = lax.iota(jnp.int32, 16)

    pltpu.sync_copy(zl_hbm, ltab_v)
    pltpu.sync_copy(zsum_hbm.at[pl.ds(sid * GROWS, GROWS)],
                    sums_sp.at[pl.ds(sid * GROWS, GROWS)])
    plsc.subcore_barrier()

    def _positions(base, j):
        return (base + (j * 16 + 1) + iota16).astype(jnp.float32)

    def _start_loads(i):
        blk = i * NW + wid
        slot = i & 1
        base = pl.multiple_of(blk * BLK, BLK)
        descs = [None, None]

        @pl.when(blk < NFULL)
        def _():
            descs[0] = pltpu.async_copy(
                feat_hbm.at[pl.ds(base, BLK)], feats[slot], fsems[slot])
            descs[1] = pltpu.async_copy(
                ids_hbm.at[pl.ds(base, BLK)], idss[slot], isems[slot])
        return descs

    pending = _start_loads(0)
    for i in range(ITERS):
        blk = i * NW + wid
        slot = i & 1
        nxt = _start_loads(i + 1) if i + 1 < ITERS else None
        cur = pending

        @pl.when(blk < NFULL)
        def _():
            cur[0].wait()
            cur[1].wait()
            base = blk * BLK
            pltpu.sync_copy(feats[slot], sums_sp.at[idss[slot]], add=True)
            for j in range(BLK // 16):
                idx = idss[slot][pl.ds(j * 16, 16)]
                plsc.store_scatter(ltab_v, [idx], _positions(base, j))
        pending = nxt

    @pl.when(wid == NW - 1)
    def _():
        base = NFULL * BLK
        pltpu.sync_copy(feat_hbm.at[pl.ds(base, TAIL)], feat_t)
        pltpu.sync_copy(ids_hbm.at[pl.ds(base, TAIL)], ids_t)
        pltpu.sync_copy(feat_t, sums_sp.at[ids_t], add=True)
        for j in range(TAIL // 16):
            idx = ids_t[pl.ds(j * 16, 16)]
            plsc.store_scatter(ltab_v, [idx], _positions(base, j))

    plsc.subcore_barrier()

    pltpu.sync_copy(sums_sp.at[pl.ds(sid * GROWS, GROWS)],
                    psums_hbm.at[cid, pl.ds(sid * GROWS, GROWS)])
    pltpu.sync_copy(ltab_v, lout_hbm.at[wid])


def _head_body(ps_ref, lt_ref, w_ref, b_ref, out_ref):
    lpos = jnp.max(lt_ref[...], axis=0)
    gi = lax.broadcasted_iota(jnp.int32, (NUM_GRAPHS, NUM_GRAPHS), 0)
    gj = lax.broadcasted_iota(jnp.int32, (NUM_GRAPHS, NUM_GRAPHS), 1)
    prev = jnp.max(jnp.where(gi < gj, lpos[:, None], 0.0), axis=0)
    cnt = jnp.maximum(lpos - prev, 1.0)[:, None]
    emb = (ps_ref[0] + ps_ref[1]) / cnt
    out_ref[...] = (
        jnp.dot(emb, w_ref[...], preferred_element_type=jnp.float32)
        + b_ref[...]
    )


def _head(psums, lout, W, b2d):
    return pl.pallas_call(
        _head_body,
        out_shape=jax.ShapeDtypeStruct((NUM_GRAPHS, DIM), jnp.float32),
    )(psums, lout, W, b2d)


@jax.jit
def kernel(node_feature, batch_ids, graph_label, W, b):
    ids = batch_ids.astype(jnp.int32)
    zsum = jnp.zeros((NUM_GRAPHS, DIM), jnp.float32)
    zl = jnp.zeros((NUM_GRAPHS,), jnp.float32)
    psums, lout = _segment_pool(node_feature, ids, zsum, zl)
    pred = _head(psums, lout, W, b.reshape(1, DIM))
    return (pred, graph_label)
